# baseline (speedup 1.0000x reference)
import functools

import jax
import jax.numpy as jnp
from jax import lax
from jax.experimental import pallas as pl
from jax.experimental.pallas import tpu as pltpu
from jax.experimental.pallas import tpu_sc as plsc

NUM_CORES = 2
NUM_SUBCORES = 16


def _make_sc_aggregate(n, npad, d, e):
    nw = NUM_CORES * NUM_SUBCORES
    epw = e // nw
    k = 80
    nch = epw // k
    rpt = npad // NUM_SUBCORES
    mesh = plsc.VectorSubcoreMesh(
        core_axis_name="c", subcore_axis_name="s",
        num_cores=NUM_CORES, num_subcores=NUM_SUBCORES)

    @functools.partial(
        pl.kernel,
        mesh=mesh,
        out_type=jax.ShapeDtypeStruct((NUM_CORES * npad, d), jnp.float32),
        scratch_types=[
            pltpu.VMEM((8, k), jnp.int32),
            pltpu.VMEM((8, k), jnp.int32),
            pltpu.VMEM((k, d), jnp.float32),
            pltpu.VMEM((k, d), jnp.float32),
            pltpu.VMEM((k, d), jnp.float32),
            pltpu.VMEM((k, d), jnp.float32),
            pltpu.VMEM_SHARED((npad, d), jnp.float32),
        ] + [pltpu.SemaphoreType.DMA] * 16,
    )
    def sc_agg(h_hbm, src_hbm, dst_hbm, z2_hbm,
               sum_hbm,
               src8, dst8, r0, r1, r2, r3, acc_sp, *sems):
        rows = [r0, r1, r2, r3]
        sem_i = sems[0:8]
        sem_g = sems[8:12]
        sem_s = sems[12:16]
        cid = lax.axis_index("c")
        sid = lax.axis_index("s")
        wid = sid * NUM_CORES + cid

        pltpu.sync_copy(z2_hbm.at[pl.ds(sid * rpt, rpt)],
                        acc_sp.at[pl.ds(sid * rpt, rpt)])
        plsc.subcore_barrier()

        ebase = wid * epw

        def fire_idx(c, sl):
            base = ebase + c * k
            pltpu.async_copy(src_hbm.at[pl.ds(base, k)], src8.at[sl],
                             sem_i[sl])
            pltpu.async_copy(dst_hbm.at[pl.ds(base, k)], dst8.at[sl],
                             sem_i[sl])

        def wait_idx(sl):
            pltpu.make_async_copy(src_hbm.at[pl.ds(0, k)], src8.at[sl],
                                  sem_i[sl]).wait()
            pltpu.make_async_copy(dst_hbm.at[pl.ds(0, k)], dst8.at[sl],
                                  sem_i[sl]).wait()

        def fire_gather(isl, rsl):
            pltpu.async_copy(h_hbm.at[pl.ds(0, k)], rows[rsl], sem_g[rsl])

        def wait_gather(rsl):
            pltpu.make_async_copy(h_hbm.at[pl.ds(0, k)], rows[rsl],
                                  sem_g[rsl]).wait()

        def fire_scatter(isl, rsl):
            pass

        def wait_scatter(rsl):
            pass

        def sub_step(c, off, wait_sc=True, fire_g=True, fire_i=True):
            if wait_sc:
                wait_scatter((off + 2) % 4)
            if fire_g:
                wait_idx((off + 2) % 8)
                fire_gather((off + 2) % 8, (off + 2) % 4)
            wait_gather(off % 4)
            fire_scatter(off % 8, off % 4)
            if fire_i:
                fire_idx(c + 6, (off + 6) % 8)

        for c0 in range(6):
            fire_idx(c0, c0)
        wait_idx(0)
        fire_gather(0, 0)
        wait_idx(1)
        fire_gather(1, 1)
        sub_step(0, 0, wait_sc=False)
        sub_step(1, 1, wait_sc=False)

        m = (nch - 6) // 8
        tail_lo = 2 + 8 * m

        def body(j, carry):
            cb = 2 + 8 * j
            for t in range(8):
                sub_step(cb + t, 2 + t)
            return carry

        lax.fori_loop(0, m, body, 0)

        for c in range(tail_lo, nch):
            sub_step(c, c % 8, wait_sc=True,
                     fire_g=(c + 2 <= nch - 1), fire_i=(c + 6 <= nch - 1))
        wait_scatter((nch - 2) % 4)
        wait_scatter((nch - 1) % 4)

        plsc.subcore_barrier()

        pltpu.sync_copy(acc_sp.at[pl.ds(sid * rpt, rpt)],
                        sum_hbm.at[pl.ds(cid * npad + sid * rpt, rpt)])

    return sc_agg


def _deg_body(d_ref, o_ref):
    dst = d_ref[...]
    lanes = lax.broadcasted_iota(jnp.int32, (1, 128), 1)
    oh_lo = (lax.rem(dst, 128) == lanes).astype(jnp.float32)
    oh_hi = (lax.div(dst, 128) == lanes).astype(jnp.float32)
    c_blk = lax.dot_general(oh_lo, oh_hi, (((0,), (0,)), ((), ())),
                            preferred_element_type=jnp.float32)

    @pl.when(pl.program_id(0) == 0)
    def _():
        o_ref[...] = jnp.zeros_like(o_ref)

    o_ref[...] += c_blk


def _tc_degrees(dst, e):
    eb = 2560
    return pl.pallas_call(
        _deg_body,
        grid=(e // eb,),
        in_specs=[pl.BlockSpec((eb, 1), lambda i: (i, 0))],
        out_specs=pl.BlockSpec((128, 128), lambda i: (0, 0)),
        out_shape=jax.ShapeDtypeStruct((128, 128), jnp.float32),
    )(dst[:, None])


def _tc_body(h_ref, p0_ref, p1_ref, d_ref, wt_ref, b_ref, o_ref):
    d = h_ref.shape[1]
    deg = jnp.maximum(d_ref[...], 1.0)
    h_n = (p0_ref[...] + p1_ref[...]) / deg
    wt = wt_ref[...]
    o_ref[...] = (
        jnp.dot(h_ref[...], wt[:d], preferred_element_type=jnp.float32)
        + jnp.dot(h_n, wt[d:], preferred_element_type=jnp.float32)
        + b_ref[...])


def _tc_linear(h, p0, p1, deg, wt, b2):
    n, d = h.shape
    out = wt.shape[1]
    blk = 1000
    return pl.pallas_call(
        _tc_body,
        grid=(n // blk,),
        in_specs=[
            pl.BlockSpec((blk, d), lambda i: (i, 0)),
            pl.BlockSpec((blk, d), lambda i: (i, 0)),
            pl.BlockSpec((blk, d), lambda i: (i, 0)),
            pl.BlockSpec((blk, 1), lambda i: (i, 0)),
            pl.BlockSpec((2 * d, out), lambda i: (0, 0)),
            pl.BlockSpec((1, out), lambda i: (0, 0)),
        ],
        out_specs=pl.BlockSpec((blk, out), lambda i: (i, 0)),
        out_shape=jax.ShapeDtypeStruct((n, out), jnp.float32),
    )(h, p0, p1, deg, wt, b2)


def kernel(h, edge_index, W, b):
    n, d = h.shape
    e = edge_index.shape[1]
    npad = ((n + 127) // 128) * 128
    src = edge_index[0]
    dst = edge_index[1]
    z2 = jnp.zeros((npad, d), jnp.float32)
    sums = _make_sc_aggregate(n, npad, d, e)(h, src, dst, z2)
    c_mat = _tc_degrees(dst, e)
    deg = c_mat.T.reshape(-1)[:n]
    wt = W.T
    return _tc_linear(h, sums[:n], sums[npad:npad + n], deg[:, None],
                      wt, b[None, :])


# speedup vs baseline: 1.2160x; 1.2160x over previous
import functools

import jax
import jax.numpy as jnp
from jax import lax
from jax.experimental import pallas as pl
from jax.experimental.pallas import tpu as pltpu
from jax.experimental.pallas import tpu_sc as plsc

NUM_CORES = 2
NUM_SUBCORES = 16


def _make_sc_aggregate(n, npad, d, e):
    nw = NUM_CORES * NUM_SUBCORES
    epw = e // nw
    k = 80
    nch = epw // k
    rpt = npad // NUM_SUBCORES
    mesh = plsc.VectorSubcoreMesh(
        core_axis_name="c", subcore_axis_name="s",
        num_cores=NUM_CORES, num_subcores=NUM_SUBCORES)

    nblk = (epw + 127) // 128
    nib = (nblk + 15) // 16

    @functools.partial(
        pl.kernel,
        mesh=mesh,
        out_type=jax.ShapeDtypeStruct((NUM_CORES * npad, d), jnp.float32),
        scratch_types=[
            pltpu.VMEM((16, 128), jnp.int32),
            pltpu.VMEM((16, 128), jnp.int32),
            pltpu.VMEM((16, 128), jnp.int32),
            pltpu.VMEM((16, 128), jnp.int32),
            pltpu.VMEM((128, d), jnp.float32),
            pltpu.VMEM((128, d), jnp.float32),
            pltpu.VMEM_SHARED((npad, d), jnp.float32),
        ] + [pltpu.SemaphoreType.DMA] * 6,
    )
    def sc_agg(h_hbm, src3_hbm, dst3_hbm, z2_hbm,
               sum_hbm,
               s0, s1, d0, d1, r0, r1, acc_sp, *sems):
        sbufs = [s0, s1]
        dbufs = [d0, d1]
        rows = [r0, r1]
        sem_i = sems[0:2]
        sem_g = sems[2:4]
        sem_s = sems[4:6]
        cid = lax.axis_index("c")
        sid = lax.axis_index("s")
        wid = sid * NUM_CORES + cid

        pltpu.sync_copy(z2_hbm.at[pl.ds(sid * rpt, rpt)],
                        acc_sp.at[pl.ds(sid * rpt, rpt)])
        plsc.subcore_barrier()

        def fire_idx(b):
            pltpu.async_copy(src3_hbm.at[wid, pl.ds(16 * b, 16)],
                             sbufs[b % 2], sem_i[b % 2])
            pltpu.async_copy(dst3_hbm.at[wid, pl.ds(16 * b, 16)],
                             dbufs[b % 2], sem_i[b % 2])

        def wait_idx(b):
            pltpu.make_async_copy(src3_hbm.at[0, pl.ds(0, 16)],
                                  sbufs[b % 2], sem_i[b % 2]).wait()
            pltpu.make_async_copy(dst3_hbm.at[0, pl.ds(0, 16)],
                                  dbufs[b % 2], sem_i[b % 2]).wait()

        def fire_gather(c):
            pltpu.async_copy(h_hbm.at[sbufs[(c // 16) % 2].at[c % 16]],
                             rows[c % 2], sem_g[c % 2])

        def wait_gather(c):
            pltpu.make_async_copy(h_hbm.at[sbufs[0].at[0]], rows[c % 2],
                                  sem_g[c % 2]).wait()

        def fire_scatter(c):
            pltpu.async_copy(rows[c % 2],
                             acc_sp.at[dbufs[(c // 16) % 2].at[c % 16]],
                             sem_s[c % 2], add=True)

        def wait_scatter(c):
            pltpu.make_async_copy(rows[c % 2], acc_sp.at[dbufs[0].at[0]],
                                  sem_s[c % 2]).wait()

        fire_idx(0)
        wait_idx(0)
        fire_idx(1)
        fire_gather(0)
        for c in range(nblk):
            wait_gather(c)
            fire_scatter(c)
            if c >= 1:
                wait_scatter(c - 1)
            if c + 1 < nblk:
                if (c + 1) % 16 == 0:
                    wait_idx((c + 1) // 16)
                fire_gather(c + 1)
            if c % 16 == 2 and c >= 16 and c // 16 + 1 < nib:
                fire_idx(c // 16 + 1)
        wait_scatter(nblk - 1)

        plsc.subcore_barrier()

        pltpu.sync_copy(acc_sp.at[pl.ds(sid * rpt, rpt)],
                        sum_hbm.at[pl.ds(cid * npad + sid * rpt, rpt)])

    return sc_agg


def _deg_body(d_ref, o_ref):
    dst = d_ref[...]
    lanes = lax.broadcasted_iota(jnp.int32, (1, 128), 1)
    oh_lo = (lax.rem(dst, 128) == lanes).astype(jnp.float32)
    oh_hi = (lax.div(dst, 128) == lanes).astype(jnp.float32)
    c_blk = lax.dot_general(oh_lo, oh_hi, (((0,), (0,)), ((), ())),
                            preferred_element_type=jnp.float32)

    @pl.when(pl.program_id(0) == 0)
    def _():
        o_ref[...] = jnp.zeros_like(o_ref)

    o_ref[...] += c_blk


def _tc_degrees(dst, e):
    eb = 2560
    return pl.pallas_call(
        _deg_body,
        grid=(e // eb,),
        in_specs=[pl.BlockSpec((eb, 1), lambda i: (i, 0))],
        out_specs=pl.BlockSpec((128, 128), lambda i: (0, 0)),
        out_shape=jax.ShapeDtypeStruct((128, 128), jnp.float32),
    )(dst[:, None])


def _tc_body(h_ref, p0_ref, p1_ref, d_ref, wt_ref, b_ref, o_ref):
    d = h_ref.shape[1]
    deg = jnp.maximum(d_ref[...], 1.0)
    h_n = (p0_ref[...] + p1_ref[...]) / deg
    wt = wt_ref[...]
    o_ref[...] = (
        jnp.dot(h_ref[...], wt[:d], preferred_element_type=jnp.float32)
        + jnp.dot(h_n, wt[d:], preferred_element_type=jnp.float32)
        + b_ref[...])


def _tc_linear(h, p0, p1, deg, wt, b2):
    n, d = h.shape
    out = wt.shape[1]
    blk = 1000
    return pl.pallas_call(
        _tc_body,
        grid=(n // blk,),
        in_specs=[
            pl.BlockSpec((blk, d), lambda i: (i, 0)),
            pl.BlockSpec((blk, d), lambda i: (i, 0)),
            pl.BlockSpec((blk, d), lambda i: (i, 0)),
            pl.BlockSpec((blk, 1), lambda i: (i, 0)),
            pl.BlockSpec((2 * d, out), lambda i: (0, 0)),
            pl.BlockSpec((1, out), lambda i: (0, 0)),
        ],
        out_specs=pl.BlockSpec((blk, out), lambda i: (i, 0)),
        out_shape=jax.ShapeDtypeStruct((n, out), jnp.float32),
    )(h, p0, p1, deg, wt, b2)


def kernel(h, edge_index, W, b):
    n, d = h.shape
    e = edge_index.shape[1]
    npad = ((n + 127) // 128) * 128
    src = edge_index[0]
    dst = edge_index[1]
    nw = NUM_CORES * NUM_SUBCORES
    epw = e // nw
    nblk = (epw + 127) // 128
    nib = (nblk + 15) // 16
    pad = nib * 16 * 128 - epw
    src3 = jnp.concatenate(
        [src.reshape(nw, epw),
         jnp.zeros((nw, pad), jnp.int32)], axis=1).reshape(nw, nib * 16, 128)
    dst3 = jnp.concatenate(
        [dst.reshape(nw, epw),
         jnp.broadcast_to(n + jnp.arange(pad, dtype=jnp.int32) % 112,
                          (nw, pad))], axis=1).reshape(nw, nib * 16, 128)
    z2 = jnp.zeros((npad, d), jnp.float32)
    sums = _make_sc_aggregate(n, npad, d, e)(h, src3, dst3, z2)
    c_mat = _tc_degrees(dst, e)
    deg = c_mat.T.reshape(-1)[:n]
    wt = W.T
    return _tc_linear(h, sums[:n], sums[npad:npad + n], deg[:, None],
                      wt, b[None, :])


# speedup vs baseline: 1.5502x; 1.2748x over previous
import functools

import jax
import jax.numpy as jnp
from jax import lax
from jax.experimental import pallas as pl
from jax.experimental.pallas import tpu as pltpu
from jax.experimental.pallas import tpu_sc as plsc

NUM_CORES = 2
NUM_SUBCORES = 16


def _make_sc_aggregate(n, npad, d, e):
    nw = NUM_CORES * NUM_SUBCORES
    epw = e // nw
    k = 80
    nch = epw // k
    rpt = npad // NUM_SUBCORES
    mesh = plsc.VectorSubcoreMesh(
        core_axis_name="c", subcore_axis_name="s",
        num_cores=NUM_CORES, num_subcores=NUM_SUBCORES)

    @functools.partial(
        pl.kernel,
        mesh=mesh,
        out_type=jax.ShapeDtypeStruct((NUM_CORES * npad, d), jnp.float32),
        scratch_types=[
            pltpu.VMEM((2, k), jnp.int32),
            pltpu.VMEM((2, k), jnp.int32),
            pltpu.VMEM((k, d), jnp.float32),
            pltpu.VMEM((k, d), jnp.float32),
            pltpu.VMEM_SHARED((npad, d), jnp.float32),
            pltpu.SemaphoreType.DMA,
            pltpu.SemaphoreType.DMA,
            pltpu.SemaphoreType.DMA,
            pltpu.SemaphoreType.DMA,
        ],
    )
    def sc_agg(h_hbm, src_hbm, dst_hbm, z2_hbm,
               sum_hbm,
               src_v, dst_v, rows_a, rows_b, acc_sp,
               sem_ia, sem_ib, sem_ga, sem_gb):
        cid = lax.axis_index("c")
        sid = lax.axis_index("s")
        wid = sid * NUM_CORES + cid

        pltpu.sync_copy(z2_hbm.at[pl.ds(sid * rpt, rpt)],
                        acc_sp.at[pl.ds(sid * rpt, rpt)])
        plsc.subcore_barrier()

        ebase = wid * epw

        def fire_idx(c, buf, sem):
            base = ebase + c * k
            d1 = pltpu.async_copy(src_hbm.at[pl.ds(base, k)],
                                  src_v.at[buf], sem)
            d2 = pltpu.async_copy(dst_hbm.at[pl.ds(base, k)],
                                  dst_v.at[buf], sem)
            return d1, d2

        def wait_idx(descs):
            descs[0].wait()
            descs[1].wait()

        def fire_gather(buf, rows, sem):
            return pltpu.async_copy(h_hbm.at[src_v.at[buf]], rows, sem)

        def scatter(buf, rows):
            pltpu.sync_copy(rows, acc_sp.at[dst_v.at[buf]], add=True)

        wait_idx(fire_idx(0, 0, sem_ia))
        fire_gather(0, rows_a, sem_ga)
        fire_idx(1, 1, sem_ib)

        def pipeline_body(j, carry):
            c = 2 * j
            pltpu.make_async_copy(src_hbm.at[pl.ds(0, k)],
                                  src_v.at[1], sem_ib).wait()
            pltpu.make_async_copy(dst_hbm.at[pl.ds(0, k)],
                                  dst_v.at[1], sem_ib).wait()
            gb = fire_gather(1, rows_b, sem_gb)
            pltpu.make_async_copy(h_hbm.at[src_v.at[0]],
                                  rows_a, sem_ga).wait()
            scatter(0, rows_a)
            wait_idx(fire_idx(c + 2, 0, sem_ia))
            fire_gather(0, rows_a, sem_ga)
            gb.wait()
            scatter(1, rows_b)
            fire_idx(c + 3, 1, sem_ib)
            return carry

        lax.fori_loop(0, (nch - 3) // 2, pipeline_body, 0)

        pltpu.make_async_copy(src_hbm.at[pl.ds(0, k)],
                              src_v.at[1], sem_ib).wait()
        pltpu.make_async_copy(dst_hbm.at[pl.ds(0, k)],
                              dst_v.at[1], sem_ib).wait()
        gb = fire_gather(1, rows_b, sem_gb)
        pltpu.make_async_copy(h_hbm.at[src_v.at[0]], rows_a, sem_ga).wait()
        scatter(0, rows_a)
        wait_idx(fire_idx(nch - 1, 0, sem_ia))
        ga = fire_gather(0, rows_a, sem_ga)
        gb.wait()
        scatter(1, rows_b)
        ga.wait()
        scatter(0, rows_a)

        plsc.subcore_barrier()

        pltpu.sync_copy(acc_sp.at[pl.ds(sid * rpt, rpt)],
                        sum_hbm.at[pl.ds(cid * npad + sid * rpt, rpt)])

    return sc_agg


def _deg_body(d_ref, o_ref):
    dst = d_ref[...]
    lanes = lax.broadcasted_iota(jnp.int32, (1, 128), 1)
    oh_lo = (lax.rem(dst, 128) == lanes).astype(jnp.float32)
    oh_hi = (lax.div(dst, 128) == lanes).astype(jnp.float32)
    c_blk = lax.dot_general(oh_lo, oh_hi, (((0,), (0,)), ((), ())),
                            preferred_element_type=jnp.float32)

    @pl.when(pl.program_id(0) == 0)
    def _():
        o_ref[...] = jnp.zeros_like(o_ref)

    o_ref[...] += c_blk


def _tc_degrees(dst, e):
    eb = 2560
    return pl.pallas_call(
        _deg_body,
        grid=(e // eb,),
        in_specs=[pl.BlockSpec((eb, 1), lambda i: (i, 0))],
        out_specs=pl.BlockSpec((128, 128), lambda i: (0, 0)),
        out_shape=jax.ShapeDtypeStruct((128, 128), jnp.float32),
    )(dst[:, None])


def _tc_body(h_ref, p0_ref, p1_ref, d_ref, wt_ref, b_ref, o_ref):
    d = h_ref.shape[1]
    deg = jnp.maximum(d_ref[...], 1.0)
    h_n = (p0_ref[...] + p1_ref[...]) / deg
    wt = wt_ref[...]
    o_ref[...] = (
        jnp.dot(h_ref[...], wt[:d], preferred_element_type=jnp.float32)
        + jnp.dot(h_n, wt[d:], preferred_element_type=jnp.float32)
        + b_ref[...])


def _tc_linear(h, p0, p1, deg, wt, b2):
    n, d = h.shape
    out = wt.shape[1]
    blk = 1000
    return pl.pallas_call(
        _tc_body,
        grid=(n // blk,),
        in_specs=[
            pl.BlockSpec((blk, d), lambda i: (i, 0)),
            pl.BlockSpec((blk, d), lambda i: (i, 0)),
            pl.BlockSpec((blk, d), lambda i: (i, 0)),
            pl.BlockSpec((blk, 1), lambda i: (i, 0)),
            pl.BlockSpec((2 * d, out), lambda i: (0, 0)),
            pl.BlockSpec((1, out), lambda i: (0, 0)),
        ],
        out_specs=pl.BlockSpec((blk, out), lambda i: (i, 0)),
        out_shape=jax.ShapeDtypeStruct((n, out), jnp.float32),
    )(h, p0, p1, deg, wt, b2)


def kernel(h, edge_index, W, b):
    n, d = h.shape
    e = edge_index.shape[1]
    npad = ((n + 127) // 128) * 128
    src = edge_index[0]
    dst = edge_index[1]
    z2 = jnp.zeros((npad, d), jnp.float32)
    sums = _make_sc_aggregate(n, npad, d, e)(h, src, dst, z2)
    c_mat = _tc_degrees(dst, e)
    deg = c_mat.T.reshape(-1)[:n]
    wt = W.T
    return _tc_linear(h, sums[:n], sums[npad:npad + n], deg[:, None],
                      wt, b[None, :])
